# baseline (speedup 1.0000x reference)
import jax
import jax.numpy as jnp
from jax import lax
from jax.experimental import pallas as pl
from jax.experimental.pallas import tpu as pltpu
from jax.experimental.pallas import tpu_sc as plsc

N_NODES = 10000
D = 128
E = 320000
NC = 2
NS = 16
NW = NC * NS
EPW = E // NW
B = 80
NCHUNK = EPW // B
NJ = D // 16
ROWBLK = 2000


def _prescale_body(s_ref, d_ref, w_ref, os_ref, od_ref):
    z = jnp.zeros((ROWBLK, D), jnp.bfloat16)
    os_ref[...] = jnp.concatenate(
        [(s_ref[...] * w_ref[...]).astype(jnp.bfloat16), z], axis=1)
    od_ref[...] = jnp.concatenate([d_ref[...].astype(jnp.bfloat16), z], axis=1)


def _prescale(h_src, h_dst, w):
    return pl.pallas_call(
        _prescale_body,
        grid=(N_NODES // ROWBLK,),
        in_specs=[
            pl.BlockSpec((ROWBLK, D), lambda i: (i, 0)),
            pl.BlockSpec((ROWBLK, D), lambda i: (i, 0)),
            pl.BlockSpec((1, D), lambda i: (0, 0)),
        ],
        out_specs=[
            pl.BlockSpec((ROWBLK, 2 * D), lambda i: (i, 0)),
            pl.BlockSpec((ROWBLK, 2 * D), lambda i: (i, 0)),
        ],
        out_shape=[
            jax.ShapeDtypeStruct((N_NODES, 2 * D), jnp.bfloat16),
            jax.ShapeDtypeStruct((N_NODES, 2 * D), jnp.bfloat16),
        ],
    )(h_src, h_dst, w.reshape(1, D))


def _sc_body(hs, hd, isrc, idst, out,
             idxs_v, idxd_v, out_v, s0, t0, s1, t1, m_v, sem0, sem1):
    c = lax.axis_index("c")
    s = lax.axis_index("s")
    wid = s * NC + c
    base = wid * EPW
    pltpu.sync_copy(isrc.at[pl.ds(base, EPW)], idxs_v)
    pltpu.sync_copy(idst.at[pl.ds(base, EPW)], idxd_v)

    def start(i, sb, tb, sem):
        pltpu.async_copy(hs.at[idxs_v.at[pl.ds(i * B, B)]], sb, sem)
        pltpu.async_copy(hd.at[idxd_v.at[pl.ds(i * B, B)]], tb, sem)

    def drain(sb, tb, sem):
        pltpu.make_async_copy(hs.at[idxs_v.at[pl.ds(0, B)]], sb, sem).wait()
        pltpu.make_async_copy(hd.at[idxd_v.at[pl.ds(0, B)]], tb, sem).wait()

    iot16 = lax.iota(jnp.int32, 16) * 16

    def compute(i, sb, tb):
        def group(g, _):
            e0 = g * 16
            accs = []
            for e in range(16):
                r = e0 + e
                parts = []
                for q in range(4):
                    sq = plsc.bitcast(sb[r, pl.ds(q * 16, 16)], jnp.bfloat16)
                    tq = plsc.bitcast(tb[r, pl.ds(q * 16, 16)], jnp.bfloat16)
                    lo, hi = plsc.unpack(sq * tq, format=plsc.PackFormat.INTERLEAVED)
                    parts.append(lo + hi)
                accs.append((parts[0] + parts[1]) + (parts[2] + parts[3]))
            for e in range(16):
                m_v[pl.ds(e * 16, 16)] = accs[e]
            cols = [plsc.load_gather(m_v, [iot16 + l]) for l in range(16)]
            while len(cols) > 1:
                cols = [a + b for a, b in zip(cols[::2], cols[1::2])]
            out_v[pl.dslice(i * B + e0, 16)] = cols[0]
            return 0

        lax.fori_loop(0, B // 16, group, 0)

    start(0, s0, t0, sem0)

    def outer(k, _):
        i0 = 2 * k
        start(i0 + 1, s1, t1, sem1)
        drain(s0, t0, sem0)
        compute(i0, s0, t0)

        @pl.when(i0 + 2 < NCHUNK)
        def _():
            start(i0 + 2, s0, t0, sem0)

        drain(s1, t1, sem1)
        compute(i0 + 1, s1, t1)
        return 0

    lax.fori_loop(0, (NCHUNK - 1) // 2, outer, 0)
    drain(s0, t0, sem0)
    compute(NCHUNK - 1, s0, t0)

    pltpu.sync_copy(out_v, out.at[pl.ds(base, EPW)])


def kernel(h_src, h_dst, edge_label_index, W):
    w = W[0]
    isrc = edge_label_index[0].astype(jnp.int32)
    idst = edge_label_index[1].astype(jnp.int32)
    hsb, hdb = _prescale(h_src, h_dst, w)
    hsw = lax.bitcast_convert_type(hsb.reshape(N_NODES, D, 2), jnp.float32)
    hdw = lax.bitcast_convert_type(hdb.reshape(N_NODES, D, 2), jnp.float32)
    mesh = plsc.VectorSubcoreMesh(
        core_axis_name="c", subcore_axis_name="s", num_cores=NC, num_subcores=NS
    )
    fn = pl.kernel(
        _sc_body,
        out_type=jax.ShapeDtypeStruct((E,), jnp.float32),
        mesh=mesh,
        compiler_params=pltpu.CompilerParams(needs_layout_passes=False),
        scratch_types=[
            pltpu.VMEM((EPW,), jnp.int32),
            pltpu.VMEM((EPW,), jnp.int32),
            pltpu.VMEM((EPW,), jnp.float32),
            pltpu.VMEM((B, D), jnp.float32),
            pltpu.VMEM((B, D), jnp.float32),
            pltpu.VMEM((B, D), jnp.float32),
            pltpu.VMEM((B, D), jnp.float32),
            pltpu.VMEM((256,), jnp.float32),
            pltpu.SemaphoreType.DMA,
            pltpu.SemaphoreType.DMA,
        ],
    )
    return fn(hsw, hdw, isrc, idst)


# speedup vs baseline: 1.8916x; 1.8916x over previous
import jax
import jax.numpy as jnp
from jax import lax
from jax.experimental import pallas as pl
from jax.experimental.pallas import tpu as pltpu
from jax.experimental.pallas import tpu_sc as plsc

N_NODES = 10000
D = 128
E = 320000
NC = 2
NS = 16
NW = NC * NS
EPW = E // NW
B = 80
NCHUNK = EPW // B
NJ = D // 16
ROWBLK = 2000


def _pack_words(lo, hi):
    l16 = lax.bitcast_convert_type(lo.astype(jnp.bfloat16), jnp.uint16).astype(jnp.uint32)
    h16 = lax.bitcast_convert_type(hi.astype(jnp.bfloat16), jnp.uint16).astype(jnp.uint32)
    w32 = lax.bitcast_convert_type(jnp.left_shift(h16, 16) | l16, jnp.float32)
    return jnp.concatenate([w32, w32], axis=1)


def _prescale_body(sa_ref, sb_ref, da_ref, db_ref, wa_ref, wb_ref, os_ref, od_ref):
    os_ref[...] = _pack_words(sa_ref[...] * wa_ref[...], sb_ref[...] * wb_ref[...])
    od_ref[...] = _pack_words(da_ref[...], db_ref[...])


def _prescale(h_src, h_dst, w):
    HD = D // 2
    rb = pl.BlockSpec((ROWBLK, HD), lambda i: (i, 0))
    wb = pl.BlockSpec((1, HD), lambda i: (0, 0))
    ob = pl.BlockSpec((ROWBLK, D), lambda i: (i, 0))
    return pl.pallas_call(
        _prescale_body,
        grid=(N_NODES // ROWBLK,),
        in_specs=[rb, rb, rb, rb, wb, wb],
        out_specs=[ob, ob],
        out_shape=[
            jax.ShapeDtypeStruct((N_NODES, D), jnp.float32),
            jax.ShapeDtypeStruct((N_NODES, D), jnp.float32),
        ],
    )(h_src[:, :HD], h_src[:, HD:], h_dst[:, :HD], h_dst[:, HD:],
      w[:HD].reshape(1, HD), w[HD:].reshape(1, HD))


def _sc_body(hs, hd, isrc, idst, out,
             idxs_v, idxd_v, out_v, s0, t0, s1, t1, m_v, sem0, sem1):
    c = lax.axis_index("c")
    s = lax.axis_index("s")
    wid = s * NC + c
    base = wid * EPW
    pltpu.sync_copy(isrc.at[pl.ds(base, EPW)], idxs_v)
    pltpu.sync_copy(idst.at[pl.ds(base, EPW)], idxd_v)

    def start(i, sb, tb, sem):
        pltpu.async_copy(hs.at[idxs_v.at[pl.ds(i * B, B)]], sb, sem)
        pltpu.async_copy(hd.at[idxd_v.at[pl.ds(i * B, B)]], tb, sem)

    def drain(sb, tb, sem):
        pltpu.make_async_copy(hs.at[idxs_v.at[pl.ds(0, B)]], sb, sem).wait()
        pltpu.make_async_copy(hd.at[idxd_v.at[pl.ds(0, B)]], tb, sem).wait()

    iot16 = lax.iota(jnp.int32, 16) * 16

    def compute(i, sb, tb):
        def group(g, _):
            e0 = g * 16
            accs = []
            for e in range(16):
                r = e0 + e
                parts = []
                for q in range(4):
                    sq = plsc.bitcast(sb[r, pl.ds(q * 16, 16)], jnp.bfloat16)
                    tq = plsc.bitcast(tb[r, pl.ds(q * 16, 16)], jnp.bfloat16)
                    lo, hi = plsc.unpack(sq * tq, format=plsc.PackFormat.INTERLEAVED)
                    parts.append(lo + hi)
                accs.append((parts[0] + parts[1]) + (parts[2] + parts[3]))
            for e in range(16):
                m_v[pl.ds(e * 16, 16)] = accs[e]
            cols = [plsc.load_gather(m_v, [iot16 + l]) for l in range(16)]
            while len(cols) > 1:
                cols = [a + b for a, b in zip(cols[::2], cols[1::2])]
            out_v[pl.dslice(i * B + e0, 16)] = cols[0]
            return 0

        lax.fori_loop(0, B // 16, group, 0)

    start(0, s0, t0, sem0)

    def outer(k, _):
        i0 = 2 * k
        start(i0 + 1, s1, t1, sem1)
        drain(s0, t0, sem0)
        compute(i0, s0, t0)

        @pl.when(i0 + 2 < NCHUNK)
        def _():
            start(i0 + 2, s0, t0, sem0)

        drain(s1, t1, sem1)
        compute(i0 + 1, s1, t1)
        return 0

    lax.fori_loop(0, (NCHUNK - 1) // 2, outer, 0)
    drain(s0, t0, sem0)
    compute(NCHUNK - 1, s0, t0)

    pltpu.sync_copy(out_v, out.at[pl.ds(base, EPW)])


def kernel(h_src, h_dst, edge_label_index, W):
    w = W[0]
    isrc = edge_label_index[0].astype(jnp.int32)
    idst = edge_label_index[1].astype(jnp.int32)
    hsw, hdw = _prescale(h_src, h_dst, w)
    mesh = plsc.VectorSubcoreMesh(
        core_axis_name="c", subcore_axis_name="s", num_cores=NC, num_subcores=NS
    )
    fn = pl.kernel(
        _sc_body,
        out_type=jax.ShapeDtypeStruct((E,), jnp.float32),
        mesh=mesh,
        compiler_params=pltpu.CompilerParams(needs_layout_passes=False),
        scratch_types=[
            pltpu.VMEM((EPW,), jnp.int32),
            pltpu.VMEM((EPW,), jnp.int32),
            pltpu.VMEM((EPW,), jnp.float32),
            pltpu.VMEM((B, D), jnp.float32),
            pltpu.VMEM((B, D), jnp.float32),
            pltpu.VMEM((B, D), jnp.float32),
            pltpu.VMEM((B, D), jnp.float32),
            pltpu.VMEM((256,), jnp.float32),
            pltpu.SemaphoreType.DMA,
            pltpu.SemaphoreType.DMA,
        ],
    )
    return fn(hsw, hdw, isrc, idst)
